# baseline (speedup 1.0000x reference)
import functools

import jax
import jax.numpy as jnp
from jax import lax
from jax.experimental import pallas as pl
from jax.experimental.pallas import tpu as pltpu
from jax.experimental.pallas import tpu_sc as plsc

_B, _L, _D, _H, _E, _K = 2, 2048, 1024, 2048, 8, 2
_N = _B * _L
_R = 256
_PR = _N * _K + _E * _R
_NBR = _PR // _R
_HT = 4
_HB = _H // _HT
_NW = 32
_GCH = 64
_CCH = 16
_SRB = 512


def _gate_body(s_ref, idx_ref, w_ref):
    scores = s_ref[...]
    lane = lax.broadcasted_iota(jnp.int32, scores.shape, 1)
    m1 = jnp.max(scores, axis=1, keepdims=True)
    i1 = jnp.min(jnp.where(scores == m1, lane, _E), axis=1, keepdims=True)
    s2 = jnp.where(lane == i1, -jnp.inf, scores)
    m2 = jnp.max(s2, axis=1, keepdims=True)
    i2 = jnp.min(jnp.where(s2 == m2, lane, _E), axis=1, keepdims=True)
    tot = jnp.maximum(m1 + m2, 1e-12)
    idx_ref[...] = jnp.where(lane == 0, i1, jnp.where(lane == 1, i2, 0))
    w_ref[...] = jnp.where(lane == 0, m1 / tot,
                           jnp.where(lane == 1, m2 / tot, 0.0))


def _gate(scores):
    return pl.pallas_call(
        _gate_body,
        out_shape=(jax.ShapeDtypeStruct((_N, _E), jnp.int32),
                   jax.ShapeDtypeStruct((_N, _E), jnp.float32)),
    )(scores)


def _moe_body(eidx_ref, valid_ref, x_ref, w_ref, wg_ref, wu_ref, wd_ref,
              out_ref):
    b = pl.program_id(0)
    h = pl.program_id(1)

    @pl.when(valid_ref[b] != 0)
    def _():
        xb = x_ref[...]
        wg = wg_ref[0].astype(jnp.bfloat16)
        wu = wu_ref[0].astype(jnp.bfloat16)
        wd = wd_ref[0].astype(jnp.bfloat16)
        gg = lax.dot_general(xb, wg, (((1,), (1,)), ((), ())),
                             preferred_element_type=jnp.float32)
        uu = lax.dot_general(xb, wu, (((1,), (1,)), ((), ())),
                             preferred_element_type=jnp.float32)
        act = (gg * jax.nn.sigmoid(gg)) * uu
        y = lax.dot_general(act.astype(jnp.bfloat16), wd,
                            (((1,), (1,)), ((), ())),
                            preferred_element_type=jnp.float32)
        y = y * w_ref[...]

        @pl.when(h == 0)
        def _():
            out_ref[...] = y

        @pl.when(h != 0)
        def _():
            out_ref[...] = out_ref[...] + y


def _grouped_swiglu(block_expert, valid, xs, w_col, Wg, Wu, Wd):
    grid_spec = pltpu.PrefetchScalarGridSpec(
        num_scalar_prefetch=2,
        grid=(_NBR, _HT),
        in_specs=[
            pl.BlockSpec((_R, _D), lambda b, h, e, v: (b, 0)),
            pl.BlockSpec((_R, 1), lambda b, h, e, v: (b, 0)),
            pl.BlockSpec((1, _HB, _D), lambda b, h, e, v: (e[b], h, 0)),
            pl.BlockSpec((1, _HB, _D), lambda b, h, e, v: (e[b], h, 0)),
            pl.BlockSpec((1, _D, _HB), lambda b, h, e, v: (e[b], 0, h)),
        ],
        out_specs=pl.BlockSpec((_R, _D), lambda b, h, e, v: (b, 0)),
    )
    return pl.pallas_call(
        _moe_body,
        grid_spec=grid_spec,
        out_shape=jax.ShapeDtypeStruct((_PR, _D), jnp.float32),
        compiler_params=pltpu.CompilerParams(
            dimension_semantics=("arbitrary", "arbitrary")),
    )(block_expert, valid, xs, w_col, Wg, Wu, Wd)


def _shared_body(x_ref, wg_ref, wu_ref, wd_ref, out_ref):
    h = pl.program_id(1)
    xb = x_ref[...].astype(jnp.bfloat16)
    wg = wg_ref[...].astype(jnp.bfloat16)
    wu = wu_ref[...].astype(jnp.bfloat16)
    wd = wd_ref[...].astype(jnp.bfloat16)
    gg = lax.dot_general(xb, wg, (((1,), (1,)), ((), ())),
                         preferred_element_type=jnp.float32)
    uu = lax.dot_general(xb, wu, (((1,), (1,)), ((), ())),
                         preferred_element_type=jnp.float32)
    act = (gg * jax.nn.sigmoid(gg)) * uu
    y = lax.dot_general(act.astype(jnp.bfloat16), wd,
                        (((1,), (1,)), ((), ())),
                        preferred_element_type=jnp.float32)

    @pl.when(h == 0)
    def _():
        out_ref[...] = y

    @pl.when(h != 0)
    def _():
        out_ref[...] = out_ref[...] + y


def _shared_swiglu(xf, Wsg, Wsu, Wsd):
    return pl.pallas_call(
        _shared_body,
        grid=(_N // _SRB, _HT),
        in_specs=[
            pl.BlockSpec((_SRB, _D), lambda b, h: (b, 0)),
            pl.BlockSpec((_HB, _D), lambda b, h: (h, 0)),
            pl.BlockSpec((_HB, _D), lambda b, h: (h, 0)),
            pl.BlockSpec((_D, _HB), lambda b, h: (0, h)),
        ],
        out_specs=pl.BlockSpec((_SRB, _D), lambda b, h: (b, 0)),
        out_shape=jax.ShapeDtypeStruct((_N, _D), jnp.float32),
        compiler_params=pltpu.CompilerParams(
            dimension_semantics=("arbitrary", "arbitrary")),
    )(xf, Wsg, Wsu, Wsd)


@functools.lru_cache(maxsize=None)
def _make_sc_gather():
    mesh = plsc.VectorSubcoreMesh(core_axis_name="c", subcore_axis_name="s")
    n_chunks = _PR // _NW // _GCH
    _DW = _D // 2

    @functools.partial(
        pl.kernel,
        mesh=mesh,
        out_type=jax.ShapeDtypeStruct((_PR, _DW), jnp.int32),
        scratch_types=[
            pltpu.VMEM((_GCH,), jnp.int32),
            pltpu.VMEM((_GCH,), jnp.int32),
            pltpu.VMEM((_GCH, _DW), jnp.int32),
            pltpu.VMEM((_GCH, _DW), jnp.int32),
            pltpu.SemaphoreType.DMA,
            pltpu.SemaphoreType.DMA,
            pltpu.SemaphoreType.DMA,
            pltpu.SemaphoreType.DMA,
        ],
    )
    def _sc_gather(xf_hbm, src_hbm, out_hbm, i0, i1, r0, r1, g0, g1, s0, s1):
        wid = lax.axis_index("s") * 2 + lax.axis_index("c")
        base = wid * (_PR // _NW)
        idx = (i0, i1)
        rows = (r0, r1)
        gsem = (g0, g1)
        ssem = (s0, s1)

        def start(c):
            bb = c & 1
            off = pl.multiple_of(base + c * _GCH, 8)
            pltpu.sync_copy(src_hbm.at[pl.ds(off, _GCH)], idx[bb])
            return pltpu.async_copy(xf_hbm.at[idx[bb]], rows[bb], gsem[bb])

        g = [start(0), start(1)]
        st = [None, None]
        for c in range(n_chunks):
            bb = c & 1
            off = pl.multiple_of(base + c * _GCH, 8)
            g[bb].wait()
            st[bb] = pltpu.async_copy(rows[bb], out_hbm.at[pl.ds(off, _GCH)],
                                      ssem[bb])
            if c + 2 < n_chunks:
                st[bb].wait()
                g[bb] = start(c + 2)
        for c in (n_chunks - 2, n_chunks - 1):
            st[c & 1].wait()

    return _sc_gather


@functools.lru_cache(maxsize=None)
def _make_sc_combine():
    mesh = plsc.VectorSubcoreMesh(core_axis_name="c", subcore_axis_name="s")
    n_chunks = _N // _NW // _CCH

    @functools.partial(
        pl.kernel,
        mesh=mesh,
        out_type=jax.ShapeDtypeStruct((_N, _D), jnp.float32),
        scratch_types=[
            pltpu.VMEM((_CCH,), jnp.int32),
            pltpu.VMEM((_CCH,), jnp.int32),
            pltpu.VMEM((_CCH,), jnp.int32),
            pltpu.VMEM((_CCH,), jnp.int32),
            pltpu.VMEM((_CCH, _D), jnp.float32),
            pltpu.VMEM((_CCH, _D), jnp.float32),
            pltpu.VMEM((_CCH, _D), jnp.float32),
            pltpu.VMEM((_CCH, _D), jnp.float32),
            pltpu.VMEM((_CCH, _D), jnp.float32),
            pltpu.VMEM((_CCH, _D), jnp.float32),
            pltpu.SemaphoreType.DMA,
            pltpu.SemaphoreType.DMA,
            pltpu.SemaphoreType.DMA,
            pltpu.SemaphoreType.DMA,
            pltpu.SemaphoreType.DMA,
            pltpu.SemaphoreType.DMA,
            pltpu.SemaphoreType.DMA,
            pltpu.SemaphoreType.DMA,
        ],
    )
    def _sc_combine(ys_hbm, ysh_hbm, pos0_hbm, pos1_hbm, out_hbm,
                    i0a, i0b, i1a, i1b, a0, a1, b0, b1, c0, c1,
                    ga0, ga1, gb0, gb1, gc0, gc1, ss0, ss1):
        wid = lax.axis_index("s") * 2 + lax.axis_index("c")
        base = wid * (_N // _NW)
        p0i = (i0a, i0b)
        p1i = (i1a, i1b)
        av = (a0, a1)
        bv = (b0, b1)
        cv = (c0, c1)
        gas = (ga0, ga1)
        gbs = (gb0, gb1)
        gcs = (gc0, gc1)
        sss = (ss0, ss1)

        def start(c):
            bb = c & 1
            t0 = pl.multiple_of(base + c * _CCH, 8)
            pltpu.sync_copy(pos0_hbm.at[pl.ds(t0, _CCH)], p0i[bb])
            pltpu.sync_copy(pos1_hbm.at[pl.ds(t0, _CCH)], p1i[bb])
            return (pltpu.async_copy(ys_hbm.at[p0i[bb]], av[bb], gas[bb]),
                    pltpu.async_copy(ys_hbm.at[p1i[bb]], bv[bb], gbs[bb]),
                    pltpu.async_copy(ysh_hbm.at[pl.ds(t0, _CCH)], cv[bb],
                                     gcs[bb]))

        g = [start(0), start(1)]
        st = [None, None]
        for c in range(n_chunks):
            bb = c & 1
            t0 = pl.multiple_of(base + c * _CCH, 8)
            for cp in g[bb]:
                cp.wait()
            a_ref, b_ref, c_ref = av[bb], bv[bb], cv[bb]

            @plsc.parallel_loop(0, _CCH * (_D // 16), unroll=8)
            def _add(i, a_ref=a_ref, b_ref=b_ref, c_ref=c_ref):
                r = lax.shift_right_logical(i, 6)
                sl = pl.ds((i & 63) * 16, 16)
                a_ref[r, sl] = a_ref[r, sl] + b_ref[r, sl] + c_ref[r, sl]

            st[bb] = pltpu.async_copy(a_ref, out_hbm.at[pl.ds(t0, _CCH)],
                                      sss[bb])
            if c + 2 < n_chunks:
                st[bb].wait()
                g[bb] = start(c + 2)
        for c in (n_chunks - 2, n_chunks - 1):
            st[c & 1].wait()

    return _sc_combine


def _dispatch_plan(idx_pad, w_pad):
    ef = idx_pad[:, :_K].reshape(-1)
    wf = w_pad[:, :_K].reshape(-1)
    oh = (ef[:, None] == jnp.arange(_E, dtype=ef.dtype)).astype(jnp.int32)
    counts = jnp.sum(oh, axis=0)
    padded = ((counts + _R - 1) // _R) * _R
    start = jnp.concatenate(
        [jnp.zeros((1,), jnp.int32),
         jnp.cumsum(padded)[:-1].astype(jnp.int32)])
    rank = jnp.sum(jnp.cumsum(oh, axis=0) * oh, axis=1) - 1
    pos = (start[ef] + rank).astype(jnp.int32)
    tok = jnp.arange(_N * _K, dtype=jnp.int32) // _K
    src = jnp.zeros((_PR,), jnp.int32).at[pos].set(tok)
    w_r = jnp.zeros((_PR,), jnp.float32).at[pos].set(wf)
    bb = jnp.arange(_NBR, dtype=jnp.int32) * _R
    be = jnp.searchsorted(start, bb, side="right").astype(jnp.int32) - 1
    valid = (bb < (start + counts)[be]).astype(jnp.int32)
    block_expert = jnp.clip(be, 0, _E - 1)
    pos2 = pos.reshape(_N, _K)
    return src, w_r, block_expert, valid, pos2[:, 0], pos2[:, 1]


def kernel(x, gate_W, Wg, Wu, Wd, Wsg, Wsu, Wsd):
    xf = x.reshape(-1, _D)
    xn = xf / jnp.maximum(jnp.linalg.norm(xf, axis=-1, keepdims=True), 1e-12)
    gwn = gate_W / jnp.maximum(
        jnp.linalg.norm(gate_W, axis=-1, keepdims=True), 1e-12)
    scores = jax.nn.sigmoid(xn @ gwn.T)
    idx_pad, w_pad = _gate(scores)
    src, w_r, block_expert, valid, pos0, pos1 = _dispatch_plan(idx_pad, w_pad)
    xhalf = lax.bitcast_convert_type(
        xf.astype(jnp.bfloat16).reshape(_N, _D // 2, 2), jnp.int32)
    xs = lax.bitcast_convert_type(
        _make_sc_gather()(xhalf, src), jnp.bfloat16).reshape(_PR, _D)
    ysh = _shared_swiglu(xf, Wsg, Wsu, Wsd)
    ys = _grouped_swiglu(block_expert, valid, xs, w_r[:, None], Wg, Wu, Wd)
    out = _make_sc_combine()(ys, ysh, pos0, pos1)
    return out.reshape(_B, _L, _D)


# speedup vs baseline: 1.9728x; 1.9728x over previous
import functools

import jax
import jax.numpy as jnp
from jax import lax
from jax.experimental import pallas as pl
from jax.experimental.pallas import tpu as pltpu
from jax.experimental.pallas import tpu_sc as plsc

_B, _L, _D, _H, _E, _K = 2, 2048, 1024, 2048, 8, 2
_N = _B * _L
_R = 256
_PR = _N * _K + _E * _R
_NBR = _PR // _R
_HT = 4
_HB = _H // _HT
_NW = 32
_GCH = 40
_CCH = 16
_SRB = 512


def _gate_body(s_ref, idx_ref, w_ref):
    scores = s_ref[...]
    lane = lax.broadcasted_iota(jnp.int32, scores.shape, 1)
    m1 = jnp.max(scores, axis=1, keepdims=True)
    i1 = jnp.min(jnp.where(scores == m1, lane, _E), axis=1, keepdims=True)
    s2 = jnp.where(lane == i1, -jnp.inf, scores)
    m2 = jnp.max(s2, axis=1, keepdims=True)
    i2 = jnp.min(jnp.where(s2 == m2, lane, _E), axis=1, keepdims=True)
    tot = jnp.maximum(m1 + m2, 1e-12)
    idx_ref[...] = jnp.where(lane == 0, i1, jnp.where(lane == 1, i2, 0))
    w_ref[...] = jnp.where(lane == 0, m1 / tot,
                           jnp.where(lane == 1, m2 / tot, 0.0))


def _gate(scores):
    return pl.pallas_call(
        _gate_body,
        out_shape=(jax.ShapeDtypeStruct((_N, _E), jnp.int32),
                   jax.ShapeDtypeStruct((_N, _E), jnp.float32)),
    )(scores)


def _moe_body(eidx_ref, valid_ref, x_ref, w_ref, wg_ref, wu_ref, wd_ref,
              out_ref):
    b = pl.program_id(0)

    @pl.when(valid_ref[b] != 0)
    def _():
        xb = x_ref[...].astype(jnp.bfloat16)
        wg = wg_ref[0].astype(jnp.bfloat16)
        wu = wu_ref[0].astype(jnp.bfloat16)
        wd = wd_ref[0].astype(jnp.bfloat16)
        gg = lax.dot_general(xb, wg, (((1,), (1,)), ((), ())),
                             preferred_element_type=jnp.float32)
        uu = lax.dot_general(xb, wu, (((1,), (1,)), ((), ())),
                             preferred_element_type=jnp.float32)
        act = (gg * jax.nn.sigmoid(gg)) * uu
        y = lax.dot_general(act.astype(jnp.bfloat16), wd,
                            (((1,), (1,)), ((), ())),
                            preferred_element_type=jnp.float32)
        out_ref[...] = y * w_ref[...]


def _grouped_swiglu(block_expert, valid, xs, w_col, Wg, Wu, Wd):
    grid_spec = pltpu.PrefetchScalarGridSpec(
        num_scalar_prefetch=2,
        grid=(_NBR,),
        in_specs=[
            pl.BlockSpec((_R, _D), lambda b, e, v: (b, 0)),
            pl.BlockSpec((_R, 1), lambda b, e, v: (b, 0)),
            pl.BlockSpec((1, _H, _D), lambda b, e, v: (e[b], 0, 0)),
            pl.BlockSpec((1, _H, _D), lambda b, e, v: (e[b], 0, 0)),
            pl.BlockSpec((1, _D, _H), lambda b, e, v: (e[b], 0, 0)),
        ],
        out_specs=pl.BlockSpec((_R, _D), lambda b, e, v: (b, 0)),
    )
    return pl.pallas_call(
        _moe_body,
        grid_spec=grid_spec,
        out_shape=jax.ShapeDtypeStruct((_PR, _D), jnp.float32),
        compiler_params=pltpu.CompilerParams(
            dimension_semantics=("arbitrary",)),
    )(block_expert, valid, xs, w_col, Wg, Wu, Wd)


def _shared_body(x_ref, wg_ref, wu_ref, wd_ref, out_ref):
    xb = x_ref[...].astype(jnp.bfloat16)
    gg = lax.dot_general(xb, wg_ref[...], (((1,), (1,)), ((), ())),
                         preferred_element_type=jnp.float32)
    uu = lax.dot_general(xb, wu_ref[...], (((1,), (1,)), ((), ())),
                         preferred_element_type=jnp.float32)
    act = (gg * jax.nn.sigmoid(gg)) * uu
    out_ref[...] = lax.dot_general(act.astype(jnp.bfloat16), wd_ref[...],
                                   (((1,), (1,)), ((), ())),
                                   preferred_element_type=jnp.float32)


def _shared_swiglu(xf, Wsg, Wsu, Wsd):
    return pl.pallas_call(
        _shared_body,
        grid=(_N // _SRB,),
        in_specs=[
            pl.BlockSpec((_SRB, _D), lambda b: (b, 0)),
            pl.BlockSpec((_H, _D), lambda b: (0, 0)),
            pl.BlockSpec((_H, _D), lambda b: (0, 0)),
            pl.BlockSpec((_D, _H), lambda b: (0, 0)),
        ],
        out_specs=pl.BlockSpec((_SRB, _D), lambda b: (b, 0)),
        out_shape=jax.ShapeDtypeStruct((_N, _D), jnp.float32),
        compiler_params=pltpu.CompilerParams(
            dimension_semantics=("arbitrary",)),
    )(xf, Wsg.astype(jnp.bfloat16), Wsu.astype(jnp.bfloat16),
      Wsd.astype(jnp.bfloat16))


@functools.lru_cache(maxsize=None)
def _make_sc_gather():
    mesh = plsc.VectorSubcoreMesh(core_axis_name="c", subcore_axis_name="s")
    n_chunks = _PR // _NW // _GCH

    @functools.partial(
        pl.kernel,
        mesh=mesh,
        out_type=jax.ShapeDtypeStruct((_PR, _D), jnp.float32),
        scratch_types=[
            pltpu.VMEM((_GCH,), jnp.int32),
            pltpu.VMEM((_GCH,), jnp.int32),
            pltpu.VMEM((_GCH, _D), jnp.float32),
            pltpu.VMEM((_GCH, _D), jnp.float32),
            pltpu.SemaphoreType.DMA,
            pltpu.SemaphoreType.DMA,
            pltpu.SemaphoreType.DMA,
            pltpu.SemaphoreType.DMA,
        ],
    )
    def _sc_gather(xf_hbm, src_hbm, out_hbm, i0, i1, r0, r1, g0, g1, s0, s1):
        wid = lax.axis_index("s") * 2 + lax.axis_index("c")
        base = wid * (_PR // _NW)
        idx = (i0, i1)
        rows = (r0, r1)
        gsem = (g0, g1)
        ssem = (s0, s1)

        def start(c):
            bb = c & 1
            off = pl.multiple_of(base + c * _GCH, 8)
            pltpu.sync_copy(src_hbm.at[pl.ds(off, _GCH)], idx[bb])
            return pltpu.async_copy(xf_hbm.at[idx[bb]], rows[bb], gsem[bb])

        g = [start(0), start(1)]
        st = [None, None]
        for c in range(n_chunks):
            bb = c & 1
            off = pl.multiple_of(base + c * _GCH, 8)
            g[bb].wait()
            st[bb] = pltpu.async_copy(rows[bb], out_hbm.at[pl.ds(off, _GCH)],
                                      ssem[bb])
            if c + 2 < n_chunks:
                st[bb].wait()
                g[bb] = start(c + 2)
        for c in (n_chunks - 2, n_chunks - 1):
            st[c & 1].wait()

    return _sc_gather


@functools.lru_cache(maxsize=None)
def _make_sc_combine():
    mesh = plsc.VectorSubcoreMesh(core_axis_name="c", subcore_axis_name="s")
    n_chunks = _N // _NW // _CCH

    @functools.partial(
        pl.kernel,
        mesh=mesh,
        out_type=jax.ShapeDtypeStruct((_N, _D), jnp.float32),
        scratch_types=[
            pltpu.VMEM((_CCH,), jnp.int32),
            pltpu.VMEM((_CCH,), jnp.int32),
            pltpu.VMEM((_CCH,), jnp.int32),
            pltpu.VMEM((_CCH,), jnp.int32),
            pltpu.VMEM((_CCH, _D), jnp.float32),
            pltpu.VMEM((_CCH, _D), jnp.float32),
            pltpu.VMEM((_CCH, _D), jnp.float32),
            pltpu.VMEM((_CCH, _D), jnp.float32),
            pltpu.VMEM((_CCH, _D), jnp.float32),
            pltpu.VMEM((_CCH, _D), jnp.float32),
            pltpu.SemaphoreType.DMA,
            pltpu.SemaphoreType.DMA,
            pltpu.SemaphoreType.DMA,
            pltpu.SemaphoreType.DMA,
            pltpu.SemaphoreType.DMA,
            pltpu.SemaphoreType.DMA,
            pltpu.SemaphoreType.DMA,
            pltpu.SemaphoreType.DMA,
        ],
    )
    def _sc_combine(ys_hbm, ysh_hbm, pos0_hbm, pos1_hbm, out_hbm,
                    i0a, i0b, i1a, i1b, a0, a1, b0, b1, c0, c1,
                    ga0, ga1, gb0, gb1, gc0, gc1, ss0, ss1):
        wid = lax.axis_index("s") * 2 + lax.axis_index("c")
        base = wid * (_N // _NW)
        p0i = (i0a, i0b)
        p1i = (i1a, i1b)
        av = (a0, a1)
        bv = (b0, b1)
        cv = (c0, c1)
        gas = (ga0, ga1)
        gbs = (gb0, gb1)
        gcs = (gc0, gc1)
        sss = (ss0, ss1)

        def start(c):
            bb = c & 1
            t0 = pl.multiple_of(base + c * _CCH, 8)
            pltpu.sync_copy(pos0_hbm.at[pl.ds(t0, _CCH)], p0i[bb])
            pltpu.sync_copy(pos1_hbm.at[pl.ds(t0, _CCH)], p1i[bb])
            return (pltpu.async_copy(ys_hbm.at[p0i[bb]], av[bb], gas[bb]),
                    pltpu.async_copy(ys_hbm.at[p1i[bb]], bv[bb], gbs[bb]),
                    pltpu.async_copy(ysh_hbm.at[pl.ds(t0, _CCH)], cv[bb],
                                     gcs[bb]))

        g = [start(0), start(1)]
        st = [None, None]
        for c in range(n_chunks):
            bb = c & 1
            t0 = pl.multiple_of(base + c * _CCH, 8)
            for cp in g[bb]:
                cp.wait()
            a_ref, b_ref, c_ref = av[bb], bv[bb], cv[bb]

            @plsc.parallel_loop(0, _CCH * (_D // 16), unroll=8)
            def _add(i, a_ref=a_ref, b_ref=b_ref, c_ref=c_ref):
                r = lax.shift_right_logical(i, 6)
                sl = pl.ds((i & 63) * 16, 16)
                a_ref[r, sl] = a_ref[r, sl] + b_ref[r, sl] + c_ref[r, sl]

            st[bb] = pltpu.async_copy(a_ref, out_hbm.at[pl.ds(t0, _CCH)],
                                      sss[bb])
            if c + 2 < n_chunks:
                st[bb].wait()
                g[bb] = start(c + 2)
        for c in (n_chunks - 2, n_chunks - 1):
            st[c & 1].wait()

    return _sc_combine


def _dispatch_plan(idx_pad, w_pad):
    ef = idx_pad[:, :_K].reshape(-1)
    wf = w_pad[:, :_K].reshape(-1)
    oh = (ef[:, None] == jnp.arange(_E, dtype=ef.dtype)).astype(jnp.int32)
    counts = jnp.sum(oh, axis=0)
    padded = ((counts + _R - 1) // _R) * _R
    start = jnp.concatenate(
        [jnp.zeros((1,), jnp.int32),
         jnp.cumsum(padded)[:-1].astype(jnp.int32)])
    rank = jnp.sum(jnp.cumsum(oh, axis=0) * oh, axis=1) - 1
    pos = (start[ef] + rank).astype(jnp.int32)
    tok = jnp.arange(_N * _K, dtype=jnp.int32) // _K
    src = jnp.zeros((_PR,), jnp.int32).at[pos].set(tok)
    w_r = jnp.zeros((_PR,), jnp.float32).at[pos].set(wf)
    bb = jnp.arange(_NBR, dtype=jnp.int32) * _R
    be = jnp.searchsorted(start, bb, side="right").astype(jnp.int32) - 1
    valid = (bb < (start + counts)[be]).astype(jnp.int32)
    block_expert = jnp.clip(be, 0, _E - 1)
    pos2 = pos.reshape(_N, _K)
    return src, w_r, block_expert, valid, pos2[:, 0], pos2[:, 1]


def kernel(x, gate_W, Wg, Wu, Wd, Wsg, Wsu, Wsd):
    xf = x.reshape(-1, _D)
    xn = xf / jnp.maximum(jnp.linalg.norm(xf, axis=-1, keepdims=True), 1e-12)
    gwn = gate_W / jnp.maximum(
        jnp.linalg.norm(gate_W, axis=-1, keepdims=True), 1e-12)
    scores = jax.nn.sigmoid(xn @ gwn.T)
    idx_pad, w_pad = _gate(scores)
    src, w_r, block_expert, valid, pos0, pos1 = _dispatch_plan(idx_pad, w_pad)
    xs = _make_sc_gather()(xf, src)
    ysh = _shared_swiglu(xf, Wsg, Wsu, Wsd)
    ys = _grouped_swiglu(block_expert, valid, xs, w_r[:, None], Wg, Wu, Wd)
    out = _make_sc_combine()(ys, ysh, pos0, pos1)
    return out.reshape(_B, _L, _D)
